# baseline (speedup 1.0000x reference)
import jax
import jax.numpy as jnp
from jax import lax
from jax.experimental import pallas as pl
from jax.experimental.pallas import tpu as pltpu
from jax.experimental.pallas import tpu_sc as plsc

MOM = 0.5
B = 16384
D = 128
N = 100000
NCL = 1000
NW = 32
BPW = B // NW
CORE = N // NW
EXT = 3136
NVI = B // 16
NVE = EXT // 16
CPYC = 224
NCH = EXT // CPYC
SUBB = 256
GB = 64
UNA = 8
TCR = 1024


def _gather_body(bank, ind_h, labs, fold_o, olab_o, idx_v, rows_v, lab_v, sem):
    wid = lax.axis_index("s") * 2 + lax.axis_index("c")
    base = wid * BPW
    pltpu.sync_copy(ind_h.at[pl.ds(base, BPW)], idx_v)
    pltpu.async_copy(bank.at[idx_v], rows_v, sem).wait()
    pltpu.sync_copy(rows_v, fold_o.at[pl.ds(base, BPW)])
    pltpu.async_copy(labs.at[idx_v], lab_v, sem).wait()
    pltpu.sync_copy(lab_v, olab_o.at[pl.ds(base, BPW)])


def _tc_body(feat, fold, cent, olab, fn2_o, nl_o, ch_o):
    i = pl.program_id(0)
    f = feat[...]
    fo = fold[...]
    fn = f / (jnp.sqrt(jnp.sum(f * f, axis=1, keepdims=True)) + 1e-10)
    fu = (1.0 - MOM) * fo + MOM * fn
    fn2 = fu / (jnp.sqrt(jnp.sum(fu * fu, axis=1, keepdims=True)) + 1e-10)
    fn2_o[...] = fn2
    sim = lax.dot_general(fn2, cent[...], (((1,), (1,)), ((), ())),
                          preferred_element_type=jnp.float32)
    mx = jnp.max(sim, axis=1, keepdims=True)
    ii = lax.broadcasted_iota(jnp.int32, sim.shape, 1)
    lbl = jnp.min(jnp.where(sim == mx, ii, jnp.int32(NCL)), axis=1)
    nl_o[0, 0, :] = lbl
    mism = jnp.sum((lbl != olab[0, 0, :]).astype(jnp.float32))
    prev = jnp.where(i == 0, 0.0, ch_o[0, 0])
    tot = prev + mism
    ch_o[0, 0] = jnp.where(i == pl.num_programs(0) - 1, tot / B, tot)


def _scatter_body(ind_h, fn2_h, nl_h, bank_h, labs_h, obank, olabs,
                  sca, nl_v, fw, idxb, lab_v, cb0, upb,
                  insem, outsem, sem):
    wid = lax.axis_index("s") * 2 + lax.axis_index("c")
    base = wid * CORE
    start = pl.multiple_of(jnp.minimum(base - lax.rem(base, 8), N - EXT), 8)
    iota = lax.iota(jnp.int32, 16)

    def in_copy(c):
        return pltpu.async_copy(bank_h.at[pl.ds(start + c * CPYC, CPYC)],
                                cb0, insem)

    def out_copy(c):
        return pltpu.async_copy(cb0,
                                obank.at[pl.ds(start + c * CPYC, CPYC)],
                                outsem)

    pltpu.sync_copy(ind_h, sca.at[pl.ds(0, B)])
    pltpu.sync_copy(nl_h, nl_v.at[pl.ds(0, B)])
    pltpu.sync_copy(labs_h.at[pl.ds(start, EXT)], lab_v)

    def passa(c, off):
        for u in range(UNA):
            j = c * UNA + u
            idx = sca[pl.ds(j * 16, 16)]
            loc = idx - start
            mask = (loc >= 0) & (loc < EXT)
            packed = (loc << 14) | (j * 16 + iota)
            plsc.store_compressed(fw.at[pl.ds(off, 16)], packed, mask=mask)
            off = off + plsc.all_reduce_population_count(mask)[0]
        return off
    m = lax.fori_loop(0, NVI // UNA, passa, jnp.int32(0))

    def compute_chunk(c, chunkb):
        lo = (c * CPYC) << 14
        hi = ((c * CPYC) + CPYC) << 14

        def filt(t, o2):
            for u in range(4):
                tv = t * 4 + u
                p = fw[pl.ds(tv * 16, 16)]
                fm = (p >= lo) & (p < hi) & ((tv * 16 + iota) < m)
                plsc.store_compressed(sca.at[pl.ds(o2, 16)], p, mask=fm)
                o2 = o2 + plsc.all_reduce_population_count(fm)[0]
            return o2
        cnt = lax.fori_loop(0, (m + 63) // 64, filt, jnp.int32(0))

        @pl.when(cnt > 0)
        def _():
            def sb_loop(sb, _):
                sboff = sb * SUBB
                for t in range(SUBB // 16):
                    pv = sca[pl.ds(sboff + t * 16, 16)]
                    idxb[pl.ds(t * 16, 16)] = pv & (B - 1)
                sbcnt = jnp.minimum(cnt - sboff, SUBB)
                for g in range(SUBB // GB):
                    @pl.when(g * GB < sbcnt)
                    def _g():
                        pltpu.async_copy(
                            fn2_h.at[idxb.at[pl.ds(g * GB, GB)]],
                            upb.at[pl.ds(g * GB, GB)], sem).wait()

                def apply(r, _2):
                    pv = sca[pl.ds(sboff + r, 16)]
                    loce = pv[0] >> 14
                    loc = loce - c * CPYC
                    iv0 = pv[0] & (B - 1)
                    for jj in range(8):
                        chunkb[loc, pl.ds(jj * 16, 16)] = (
                            upb[r, pl.ds(jj * 16, 16)])
                    lane = loce & 15
                    basel = loce - lane
                    lv = nl_v[pl.ds(iv0, 16)]
                    cur = lab_v[pl.ds(basel, 16)]
                    lab_v[pl.ds(basel, 16)] = jnp.where(iota == lane,
                                                        lv[0], cur)
                    return 0
                lax.fori_loop(0, sbcnt, apply, 0)
                return 0
            lax.fori_loop(0, (cnt + SUBB - 1) // SUBB, sb_loop, 0)

    for c in range(NCH):
        in_copy(c).wait()
        compute_chunk(c, cb0)
        out_copy(c).wait()
    pltpu.sync_copy(lab_v, olabs.at[pl.ds(start, EXT)])


def kernel(feature, ind, feature_bank, cluster_centroids, cluster_label_bank):
    ind32 = ind.astype(jnp.int32)
    mesh = plsc.VectorSubcoreMesh(core_axis_name="c", subcore_axis_name="s")

    fold, olab = pl.kernel(
        _gather_body,
        out_type=[jax.ShapeDtypeStruct((B, D), jnp.float32),
                  jax.ShapeDtypeStruct((B,), jnp.int32)],
        mesh=mesh,
        scratch_types=[pltpu.VMEM((BPW,), jnp.int32),
                       pltpu.VMEM((BPW, D), jnp.float32),
                       pltpu.VMEM((BPW,), jnp.int32),
                       pltpu.SemaphoreType.DMA],
    )(feature_bank, ind32, cluster_label_bank)

    fn2, nl3, ch = pl.pallas_call(
        _tc_body,
        out_shape=[jax.ShapeDtypeStruct((B, D), jnp.float32),
                   jax.ShapeDtypeStruct((B // TCR, 1, TCR), jnp.int32),
                   jax.ShapeDtypeStruct((1, 1), jnp.float32)],
        grid=(B // TCR,),
        in_specs=[pl.BlockSpec((TCR, D), lambda i: (i, 0)),
                  pl.BlockSpec((TCR, D), lambda i: (i, 0)),
                  pl.BlockSpec((NCL, D), lambda i: (0, 0)),
                  pl.BlockSpec((1, 1, TCR), lambda i: (i, 0, 0))],
        out_specs=[pl.BlockSpec((TCR, D), lambda i: (i, 0)),
                   pl.BlockSpec((1, 1, TCR), lambda i: (i, 0, 0)),
                   pl.BlockSpec(memory_space=pltpu.SMEM)],
    )(feature, fold, cluster_centroids, olab.reshape(B // TCR, 1, TCR))
    newlabel = nl3.reshape(B)

    new_bank, new_labels = pl.kernel(
        _scatter_body,
        out_type=[jax.ShapeDtypeStruct((N, D), jnp.float32),
                  jax.ShapeDtypeStruct((N,), jnp.int32)],
        mesh=mesh,
        scratch_types=[pltpu.VMEM((B + 64,), jnp.int32),
                       pltpu.VMEM((B + 16,), jnp.int32),
                       pltpu.VMEM((B + 64,), jnp.int32),
                       pltpu.VMEM((SUBB,), jnp.int32),
                       pltpu.VMEM((EXT,), jnp.int32),
                       pltpu.VMEM((CPYC, D), jnp.float32),
                       pltpu.VMEM((SUBB, D), jnp.float32),
                       pltpu.SemaphoreType.DMA,
                       pltpu.SemaphoreType.DMA,
                       pltpu.SemaphoreType.DMA],
        compiler_params=pltpu.CompilerParams(needs_layout_passes=False),
    )(ind32, fn2, newlabel, feature_bank, cluster_label_bank)

    return (ch.reshape(()), fn2, new_bank, new_labels)
